# baseline (speedup 1.0000x reference)
import functools

import jax
import jax.numpy as jnp
from jax import lax
from jax.experimental import pallas as pl
from jax.experimental.pallas import tpu as pltpu
from jax.experimental.pallas import tpu_sc as plsc

N = 10000
E = 320000
D = 128
H = 128

NC = 2
NS = 16
NW = NC * NS
EPW = E // NW
CS = 128
NCH = 80
PAD = NCH * CS - EPW
NSTG = 2
HS = NCH // NSTG
TPAD = 8
TR = N + TPAD

ZR = 624
TAIL = N - NS * ZR
TAIL_OFF = NS * ZR


def _striped_copy(src_ref, dst_ref, sid):
    pltpu.sync_copy(src_ref.at[pl.ds(sid * ZR, ZR)],
                    dst_ref.at[pl.ds(sid * ZR, ZR)])

    @pl.when(sid == 0)
    def _():
        pltpu.sync_copy(src_ref.at[pl.ds(TAIL_OFF, TAIL)],
                        dst_ref.at[pl.ds(TAIL_OFF, TAIL)])


@functools.lru_cache(maxsize=None)
def _sc_scatter_kernel():
    mesh = plsc.VectorSubcoreMesh(core_axis_name="c", subcore_axis_name="s")
    return functools.partial(
        pl.kernel,
        mesh=mesh,
        out_type=jax.ShapeDtypeStruct((NC, N, H), jnp.float32),
        scratch_types=[
            pltpu.VMEM((HS, CS), jnp.int32),
            pltpu.VMEM((HS, CS), jnp.int32),
            pltpu.VMEM((CS, H), jnp.float32),
            pltpu.VMEM((CS, H), jnp.float32),
            pltpu.SemaphoreType.DMA,
            pltpu.SemaphoreType.DMA,
            pltpu.VMEM_SHARED((N, H), jnp.float32),
        ],
    )(_sc_scatter_body)


def _sc_scatter_body(y_hbm, src_hbm, dst_hbm, zeros_hbm, out_hbm,
                     src_v, dst_v, rows0, rows1, sem0, sem1, acc_sh):
    cid = lax.axis_index("c")
    sid = lax.axis_index("s")
    wid = sid * NC + cid
    _striped_copy(zeros_hbm, acc_sh, sid)
    plsc.subcore_barrier()

    bufs = (rows0, rows1)
    sems = (sem0, sem1)

    for s in range(NSTG):
        pid = wid * NSTG + s
        pltpu.sync_copy(src_hbm.at[pid], src_v)
        pltpu.sync_copy(dst_hbm.at[pid], dst_v)
        pltpu.async_copy(y_hbm.at[src_v.at[0]], rows0, sem0)
        pltpu.async_copy(y_hbm.at[src_v.at[1]], rows1, sem1)

        @pl.loop(0, HS, step=2)
        def _(j):
            for b in range(2):
                jj = j + b
                buf = bufs[b]
                sem = sems[b]

                pltpu.make_async_copy(y_hbm.at[src_v.at[jj]], buf, sem).wait()
                pltpu.sync_copy(buf, acc_sh.at[dst_v.at[jj]], add=True)

                @pl.when(jj + 2 < HS)
                def _():
                    pltpu.async_copy(y_hbm.at[src_v.at[jj + 2]], buf, sem)

    plsc.subcore_barrier()
    _striped_copy(acc_sh, out_hbm.at[cid], sid)


BN = 1000


def _mm_body(x_ref, w_ref, o_ref):
    o_ref[...] = jnp.dot(x_ref[...], w_ref[...],
                         preferred_element_type=jnp.float32)


def _tc_matmul(x, w):
    m, k = x.shape
    n = w.shape[1]
    return pl.pallas_call(
        _mm_body,
        grid=(m // BN,),
        in_specs=[pl.BlockSpec((BN, k), lambda i: (i, 0)),
                  pl.BlockSpec((k, n), lambda i: (0, 0))],
        out_specs=pl.BlockSpec((BN, n), lambda i: (i, 0)),
        out_shape=jax.ShapeDtypeStruct((m, n), jnp.float32),
    )(x, w)


def _scale_body(dp_ref, xw_ref, y_ref, dis_ref):
    deg = 1.0 + dp_ref[0, :, 0:1] + dp_ref[1, :, 0:1]
    dis = lax.rsqrt(deg)
    dis_ref[...] = dis
    y_ref[...] = xw_ref[...] * dis


def _tc_scale(degp, xw):
    return pl.pallas_call(
        _scale_body,
        grid=(N // BN,),
        in_specs=[pl.BlockSpec((NC, BN, H), lambda i: (0, i, 0)),
                  pl.BlockSpec((BN, H), lambda i: (i, 0))],
        out_specs=[pl.BlockSpec((BN, H), lambda i: (i, 0)),
                   pl.BlockSpec((BN, 1), lambda i: (i, 0))],
        out_shape=[jax.ShapeDtypeStruct((N, H), jnp.float32),
                   jax.ShapeDtypeStruct((N, 1), jnp.float32)],
    )(degp, xw)


def _ln(h, g_ref, be_ref):
    m = jnp.mean(h, axis=-1, keepdims=True)
    c = h - m
    v = jnp.mean(c * c, axis=-1, keepdims=True)
    return c * lax.rsqrt(v + 1e-5) * g_ref[...] + be_ref[...]


def _mid_body(p_ref, y_ref, dis_ref, b_ref, g_ref, be_ref, w_ref,
              x1_ref, y2_ref):
    agg = p_ref[0] + p_ref[1] + y_ref[...]
    h = agg * dis_ref[...] + b_ref[...]
    x1 = jnp.maximum(_ln(h, g_ref, be_ref), 0.0)
    x1_ref[...] = x1
    y2_ref[...] = jnp.dot(x1, w_ref[...],
                          preferred_element_type=jnp.float32) * dis_ref[...]


def _tc_mid(p, y1, dis, b1, g1, be1, w2):
    return pl.pallas_call(
        _mid_body,
        grid=(N // BN,),
        in_specs=[pl.BlockSpec((NC, BN, H), lambda i: (0, i, 0)),
                  pl.BlockSpec((BN, H), lambda i: (i, 0)),
                  pl.BlockSpec((BN, 1), lambda i: (i, 0)),
                  pl.BlockSpec((1, H), lambda i: (0, 0)),
                  pl.BlockSpec((1, H), lambda i: (0, 0)),
                  pl.BlockSpec((1, H), lambda i: (0, 0)),
                  pl.BlockSpec((H, H), lambda i: (0, 0))],
        out_specs=[pl.BlockSpec((BN, H), lambda i: (i, 0)),
                   pl.BlockSpec((BN, H), lambda i: (i, 0))],
        out_shape=[jax.ShapeDtypeStruct((N, H), jnp.float32),
                   jax.ShapeDtypeStruct((N, H), jnp.float32)],
    )(p, y1, dis, b1, g1, be1, w2)


def _final_body(q_ref, y_ref, dis_ref, b_ref, g_ref, be_ref, x1_ref,
                wf_ref, bf_ref, o_ref):
    agg = q_ref[0] + q_ref[1] + y_ref[...]
    h = agg * dis_ref[...] + b_ref[...]
    x2 = jnp.maximum(_ln(h, g_ref, be_ref), 0.0) + x1_ref[...]
    o_ref[...] = jnp.dot(x2, wf_ref[...],
                         preferred_element_type=jnp.float32) + bf_ref[...]


def _tc_final(q, y2, dis, b2, g2, be2, x1, wf, bf):
    return pl.pallas_call(
        _final_body,
        grid=(N // BN,),
        in_specs=[pl.BlockSpec((NC, BN, H), lambda i: (0, i, 0)),
                  pl.BlockSpec((BN, H), lambda i: (i, 0)),
                  pl.BlockSpec((BN, 1), lambda i: (i, 0)),
                  pl.BlockSpec((1, H), lambda i: (0, 0)),
                  pl.BlockSpec((1, H), lambda i: (0, 0)),
                  pl.BlockSpec((1, H), lambda i: (0, 0)),
                  pl.BlockSpec((BN, H), lambda i: (i, 0)),
                  pl.BlockSpec((H, 1), lambda i: (0, 0)),
                  pl.BlockSpec((1, 1), lambda i: (0, 0))],
        out_specs=pl.BlockSpec((BN, 1), lambda i: (i, 0)),
        out_shape=jax.ShapeDtypeStruct((N, 1), jnp.float32),
    )(q, y2, dis, b2, g2, be2, x1, wf, bf)


def _plane(idx, pad_val):
    pad = jnp.full((NW, PAD), pad_val, jnp.int32)
    return jnp.concatenate([idx.reshape(NW, EPW), pad],
                           axis=1).reshape(NW * NSTG, HS, CS)


def kernel(x, edge_index, W1, b1, g1, be1, W2, b2, g2, be2, Wf, bf):
    src_i32 = edge_index[0].astype(jnp.int32)
    dst_i32 = edge_index[1].astype(jnp.int32)
    src3 = _plane(src_i32, N)
    dst3 = _plane(dst_i32, 0)
    deg_src3 = _plane(dst_i32, N)
    zeros_rows = jnp.zeros((N, H), jnp.float32)
    zpad = jnp.zeros((TPAD, H), jnp.float32)
    onesz = jnp.concatenate([jnp.ones((N, H), jnp.float32), zpad])

    b1r = b1.reshape(1, H)
    g1r = g1.reshape(1, H)
    be1r = be1.reshape(1, H)
    b2r = b2.reshape(1, H)
    g2r = g2.reshape(1, H)
    be2r = be2.reshape(1, H)
    bfr = bf.reshape(1, 1)

    sc_scatter = _sc_scatter_kernel()

    degp = sc_scatter(onesz, deg_src3, dst3, zeros_rows)
    xw1 = _tc_matmul(x, W1)
    y1, dis = _tc_scale(degp, xw1)

    p = sc_scatter(jnp.concatenate([y1, zpad]), src3, dst3, zeros_rows)
    x1, y2 = _tc_mid(p, y1, dis, b1r, g1r, be1r, W2)

    q = sc_scatter(jnp.concatenate([y2, zpad]), src3, dst3, zeros_rows)
    return _tc_final(q, y2, dis, b2r, g2r, be2r, x1, Wf, bfr)


# speedup vs baseline: 1.4285x; 1.4285x over previous
import functools

import jax
import jax.numpy as jnp
from jax import lax
from jax.experimental import pallas as pl
from jax.experimental.pallas import tpu as pltpu
from jax.experimental.pallas import tpu_sc as plsc

N = 10000
E = 320000
D = 128
H = 128

NC = 2
NS = 16
NW = NC * NS
EPW = E // NW
CS = 128
NCH = 80
PAD = NCH * CS - EPW
NSTG = 2
HS = NCH // NSTG
TPAD = 8
TR = N + TPAD

NV = (EPW + PAD) // 16
CNT = N + 16

ZR = 624
TAIL = N - NS * ZR
TAIL_OFF = NS * ZR


def _striped_copy(src_ref, dst_ref, sid):
    pltpu.sync_copy(src_ref.at[pl.ds(sid * ZR, ZR)],
                    dst_ref.at[pl.ds(sid * ZR, ZR)])

    @pl.when(sid == 0)
    def _():
        pltpu.sync_copy(src_ref.at[pl.ds(TAIL_OFF, TAIL)],
                        dst_ref.at[pl.ds(TAIL_OFF, TAIL)])


@functools.lru_cache(maxsize=None)
def _sc_degree_kernel():
    mesh = plsc.VectorSubcoreMesh(core_axis_name="c", subcore_axis_name="s")
    return functools.partial(
        pl.kernel,
        mesh=mesh,
        out_type=jax.ShapeDtypeStruct((NW, CNT), jnp.float32),
        scratch_types=[
            pltpu.VMEM((NV * 16,), jnp.int32),
            pltpu.VMEM((CNT,), jnp.float32),
        ],
        compiler_params=pltpu.CompilerParams(needs_layout_passes=False),
    )(_sc_degree_body)


def _sc_degree_body(dst_hbm, zeros_hbm, out_hbm, idx_v, cnt_v):
    cid = lax.axis_index("c")
    sid = lax.axis_index("s")
    wid = sid * NC + cid
    pltpu.sync_copy(zeros_hbm, cnt_v)
    pltpu.sync_copy(dst_hbm.at[wid], idx_v)
    ones = jnp.ones((16,), jnp.float32)

    @pl.loop(0, NV)
    def _(r):
        idx = idx_v[pl.ds(r * 16, 16)]
        plsc.addupdate_scatter(cnt_v, [idx], ones)

    pltpu.sync_copy(cnt_v, out_hbm.at[wid])


@functools.lru_cache(maxsize=None)
def _sc_scatter_kernel():
    mesh = plsc.VectorSubcoreMesh(core_axis_name="c", subcore_axis_name="s")
    return functools.partial(
        pl.kernel,
        mesh=mesh,
        out_type=jax.ShapeDtypeStruct((NC, N, H), jnp.float32),
        scratch_types=[
            pltpu.VMEM((HS, CS), jnp.int32),
            pltpu.VMEM((HS, CS), jnp.int32),
            pltpu.VMEM((CS, H), jnp.float32),
            pltpu.VMEM((CS, H), jnp.float32),
            pltpu.SemaphoreType.DMA,
            pltpu.SemaphoreType.DMA,
            pltpu.VMEM_SHARED((N, H), jnp.float32),
        ],
    )(_sc_scatter_body)


def _sc_scatter_body(y_hbm, src_hbm, dst_hbm, zeros_hbm, out_hbm,
                     src_v, dst_v, rows0, rows1, sem0, sem1, acc_sh):
    cid = lax.axis_index("c")
    sid = lax.axis_index("s")
    wid = sid * NC + cid
    _striped_copy(zeros_hbm, acc_sh, sid)
    plsc.subcore_barrier()

    bufs = (rows0, rows1)
    sems = (sem0, sem1)

    for s in range(NSTG):
        pid = wid * NSTG + s
        pltpu.sync_copy(src_hbm.at[pid], src_v)
        pltpu.sync_copy(dst_hbm.at[pid], dst_v)
        pltpu.async_copy(y_hbm.at[src_v.at[0]], rows0, sem0)
        pltpu.async_copy(y_hbm.at[src_v.at[1]], rows1, sem1)

        @pl.loop(0, HS, step=2)
        def _(j):
            for b in range(2):
                jj = j + b
                buf = bufs[b]
                sem = sems[b]

                pltpu.make_async_copy(y_hbm.at[src_v.at[jj]], buf, sem).wait()
                pltpu.sync_copy(buf, acc_sh.at[dst_v.at[jj]], add=True)

                @pl.when(jj + 2 < HS)
                def _():
                    pltpu.async_copy(y_hbm.at[src_v.at[jj + 2]], buf, sem)

    plsc.subcore_barrier()
    _striped_copy(acc_sh, out_hbm.at[cid], sid)


BN = 1000


def _mm_body(x_ref, w_ref, o_ref):
    o_ref[...] = jnp.dot(x_ref[...], w_ref[...],
                         preferred_element_type=jnp.float32)


def _tc_matmul(x, w):
    m, k = x.shape
    n = w.shape[1]
    return pl.pallas_call(
        _mm_body,
        grid=(m // BN,),
        in_specs=[pl.BlockSpec((BN, k), lambda i: (i, 0)),
                  pl.BlockSpec((k, n), lambda i: (0, 0))],
        out_specs=pl.BlockSpec((BN, n), lambda i: (i, 0)),
        out_shape=jax.ShapeDtypeStruct((m, n), jnp.float32),
    )(x, w)


def _scale_body(dp_ref, xw_ref, y_ref, dis_ref):
    deg = 1.0 + jnp.sum(dp_ref[...], axis=1)[:, None]
    dis = lax.rsqrt(deg)
    dis_ref[...] = dis
    y_ref[...] = xw_ref[...] * dis


def _tc_scale(degp, xw):
    return pl.pallas_call(
        _scale_body,
        grid=(N // BN,),
        in_specs=[pl.BlockSpec((BN, NW), lambda i: (i, 0)),
                  pl.BlockSpec((BN, H), lambda i: (i, 0))],
        out_specs=[pl.BlockSpec((BN, H), lambda i: (i, 0)),
                   pl.BlockSpec((BN, 1), lambda i: (i, 0))],
        out_shape=[jax.ShapeDtypeStruct((N, H), jnp.float32),
                   jax.ShapeDtypeStruct((N, 1), jnp.float32)],
    )(degp, xw)


def _ln(h, g_ref, be_ref):
    m = jnp.mean(h, axis=-1, keepdims=True)
    c = h - m
    v = jnp.mean(c * c, axis=-1, keepdims=True)
    return c * lax.rsqrt(v + 1e-5) * g_ref[...] + be_ref[...]


def _mid_body(p_ref, y_ref, dis_ref, b_ref, g_ref, be_ref, w_ref,
              x1_ref, y2_ref):
    agg = p_ref[0] + p_ref[1] + y_ref[...]
    h = agg * dis_ref[...] + b_ref[...]
    x1 = jnp.maximum(_ln(h, g_ref, be_ref), 0.0)
    x1_ref[...] = x1
    y2_ref[...] = jnp.dot(x1, w_ref[...],
                          preferred_element_type=jnp.float32) * dis_ref[...]


def _tc_mid(p, y1, dis, b1, g1, be1, w2):
    return pl.pallas_call(
        _mid_body,
        grid=(N // BN,),
        in_specs=[pl.BlockSpec((NC, BN, H), lambda i: (0, i, 0)),
                  pl.BlockSpec((BN, H), lambda i: (i, 0)),
                  pl.BlockSpec((BN, 1), lambda i: (i, 0)),
                  pl.BlockSpec((1, H), lambda i: (0, 0)),
                  pl.BlockSpec((1, H), lambda i: (0, 0)),
                  pl.BlockSpec((1, H), lambda i: (0, 0)),
                  pl.BlockSpec((H, H), lambda i: (0, 0))],
        out_specs=[pl.BlockSpec((BN, H), lambda i: (i, 0)),
                   pl.BlockSpec((BN, H), lambda i: (i, 0))],
        out_shape=[jax.ShapeDtypeStruct((N, H), jnp.float32),
                   jax.ShapeDtypeStruct((N, H), jnp.float32)],
    )(p, y1, dis, b1, g1, be1, w2)


def _final_body(q_ref, y_ref, dis_ref, b_ref, g_ref, be_ref, x1_ref,
                wf_ref, bf_ref, o_ref):
    agg = q_ref[0] + q_ref[1] + y_ref[...]
    h = agg * dis_ref[...] + b_ref[...]
    x2 = jnp.maximum(_ln(h, g_ref, be_ref), 0.0) + x1_ref[...]
    o_ref[...] = jnp.dot(x2, wf_ref[...],
                         preferred_element_type=jnp.float32) + bf_ref[...]


def _tc_final(q, y2, dis, b2, g2, be2, x1, wf, bf):
    return pl.pallas_call(
        _final_body,
        grid=(N // BN,),
        in_specs=[pl.BlockSpec((NC, BN, H), lambda i: (0, i, 0)),
                  pl.BlockSpec((BN, H), lambda i: (i, 0)),
                  pl.BlockSpec((BN, 1), lambda i: (i, 0)),
                  pl.BlockSpec((1, H), lambda i: (0, 0)),
                  pl.BlockSpec((1, H), lambda i: (0, 0)),
                  pl.BlockSpec((1, H), lambda i: (0, 0)),
                  pl.BlockSpec((BN, H), lambda i: (i, 0)),
                  pl.BlockSpec((H, 1), lambda i: (0, 0)),
                  pl.BlockSpec((1, 1), lambda i: (0, 0))],
        out_specs=pl.BlockSpec((BN, 1), lambda i: (i, 0)),
        out_shape=jax.ShapeDtypeStruct((N, 1), jnp.float32),
    )(q, y2, dis, b2, g2, be2, x1, wf, bf)


def _plane(idx, pad_val):
    pad = jnp.full((NW, PAD), pad_val, jnp.int32)
    return jnp.concatenate([idx.reshape(NW, EPW), pad],
                           axis=1).reshape(NW * NSTG, HS, CS)


def kernel(x, edge_index, W1, b1, g1, be1, W2, b2, g2, be2, Wf, bf):
    src_i32 = edge_index[0].astype(jnp.int32)
    dst_i32 = edge_index[1].astype(jnp.int32)
    src3 = _plane(src_i32, N)
    dst3 = _plane(dst_i32, 0)
    dstv = jnp.concatenate(
        [dst_i32.reshape(NW, EPW), jnp.full((NW, PAD), N, jnp.int32)],
        axis=1)
    zeros_rows = jnp.zeros((N, H), jnp.float32)
    zeros_cnt = jnp.zeros((CNT,), jnp.float32)
    zpad = jnp.zeros((TPAD, H), jnp.float32)

    b1r = b1.reshape(1, H)
    g1r = g1.reshape(1, H)
    be1r = be1.reshape(1, H)
    b2r = b2.reshape(1, H)
    g2r = g2.reshape(1, H)
    be2r = be2.reshape(1, H)
    bfr = bf.reshape(1, 1)

    sc_scatter = _sc_scatter_kernel()
    sc_degree = _sc_degree_kernel()

    degp = sc_degree(dstv, zeros_cnt)[:, :N].T
    xw1 = _tc_matmul(x, W1)
    y1, dis = _tc_scale(degp, xw1)

    p = sc_scatter(jnp.concatenate([y1, zpad]), src3, dst3, zeros_rows)
    x1, y2 = _tc_mid(p, y1, dis, b1r, g1r, be1r, W2)

    q = sc_scatter(jnp.concatenate([y2, zpad]), src3, dst3, zeros_rows)
    return _tc_final(q, y2, dis, b2r, g2r, be2r, x1, Wf, bfr)


# speedup vs baseline: 4.0126x; 2.8090x over previous
import functools

import jax
import jax.numpy as jnp
from jax import lax
from jax.experimental import pallas as pl
from jax.experimental.pallas import tpu as pltpu
from jax.experimental.pallas import tpu_sc as plsc

N = 10000
E = 320000
D = 128
H = 128

NC = 2
NS = 16
NW = NC * NS
EPW = E // NW
CS = 80
NCH = 125
NSTG = 5
HS = NCH // NSTG
NBUF = 4
PAD = 240

NV = (EPW + PAD) // 16
CNT = N + 16

ZR = 624
TAIL = N - NS * ZR
TAIL_OFF = NS * ZR


def _striped_copy(src_ref, dst_ref, sid):
    pltpu.sync_copy(src_ref.at[pl.ds(sid * ZR, ZR)],
                    dst_ref.at[pl.ds(sid * ZR, ZR)])

    @pl.when(sid == 0)
    def _():
        pltpu.sync_copy(src_ref.at[pl.ds(TAIL_OFF, TAIL)],
                        dst_ref.at[pl.ds(TAIL_OFF, TAIL)])


@functools.lru_cache(maxsize=None)
def _sc_degree_kernel():
    mesh = plsc.VectorSubcoreMesh(core_axis_name="c", subcore_axis_name="s")
    return functools.partial(
        pl.kernel,
        mesh=mesh,
        out_type=jax.ShapeDtypeStruct((NW, CNT), jnp.float32),
        scratch_types=[
            pltpu.VMEM((NV * 16,), jnp.int32),
            pltpu.VMEM((CNT,), jnp.float32),
        ],
        compiler_params=pltpu.CompilerParams(needs_layout_passes=False),
    )(_sc_degree_body)


def _sc_degree_body(dst_hbm, zeros_hbm, out_hbm, idx_v, cnt_v):
    cid = lax.axis_index("c")
    sid = lax.axis_index("s")
    wid = sid * NC + cid
    pltpu.sync_copy(zeros_hbm, cnt_v)
    pltpu.sync_copy(dst_hbm.at[wid], idx_v)
    ones = jnp.ones((16,), jnp.float32)

    @pl.loop(0, NV)
    def _(r):
        idx = idx_v[pl.ds(r * 16, 16)]
        plsc.addupdate_scatter(cnt_v, [idx], ones)

    pltpu.sync_copy(cnt_v, out_hbm.at[wid])


@functools.lru_cache(maxsize=None)
def _sc_scatter_kernel():
    mesh = plsc.VectorSubcoreMesh(core_axis_name="c", subcore_axis_name="s")
    return functools.partial(
        pl.kernel,
        mesh=mesh,
        out_type=jax.ShapeDtypeStruct((NC, N, H), jnp.float32),
        scratch_types=[
            pltpu.VMEM((HS, CS), jnp.int32),
            pltpu.VMEM((HS, CS), jnp.int32),
        ] + [pltpu.VMEM((CS, H), jnp.float32)] * NBUF
          + [pltpu.SemaphoreType.DMA] * NBUF
          + [pltpu.VMEM_SHARED((N, H), jnp.float32)],
    )(_sc_scatter_body)


def _sc_scatter_body(y_hbm, src_hbm, dst_hbm, zeros_hbm, out_hbm,
                     src_v, dst_v, *rest):
    bufs = rest[:NBUF]
    sems = rest[NBUF:2 * NBUF]
    acc_sh = rest[2 * NBUF]
    cid = lax.axis_index("c")
    sid = lax.axis_index("s")
    wid = sid * NC + cid
    _striped_copy(zeros_hbm, acc_sh, sid)
    plsc.subcore_barrier()

    for s in range(NSTG):
        pid = wid * NSTG + s
        pltpu.sync_copy(src_hbm.at[pid], src_v)
        pltpu.sync_copy(dst_hbm.at[pid], dst_v)
        for b in range(NBUF):
            pltpu.async_copy(y_hbm.at[src_v.at[b]], bufs[b], sems[b])

        @pl.loop(0, HS - 1, step=NBUF)
        def _(j):
            for b in range(NBUF):
                jj = j + b
                buf = bufs[b]
                sem = sems[b]

                pltpu.make_async_copy(y_hbm.at[src_v.at[jj]], buf, sem).wait()
                pltpu.sync_copy(buf, acc_sh.at[dst_v.at[jj]], add=True)

                @pl.when(jj + NBUF < HS)
                def _():
                    pltpu.async_copy(y_hbm.at[src_v.at[jj + NBUF]], buf, sem)

        pltpu.make_async_copy(y_hbm.at[src_v.at[HS - 1]], bufs[0],
                              sems[0]).wait()
        pltpu.sync_copy(bufs[0], acc_sh.at[dst_v.at[HS - 1]], add=True)

    plsc.subcore_barrier()
    _striped_copy(acc_sh, out_hbm.at[cid], sid)


BN = 1000


def _mm_body(x_ref, w_ref, o_ref):
    o_ref[...] = jnp.dot(x_ref[...], w_ref[...],
                         preferred_element_type=jnp.float32)


def _tc_matmul(x, w):
    m, k = x.shape
    n = w.shape[1]
    return pl.pallas_call(
        _mm_body,
        grid=(m // BN,),
        in_specs=[pl.BlockSpec((BN, k), lambda i: (i, 0)),
                  pl.BlockSpec((k, n), lambda i: (0, 0))],
        out_specs=pl.BlockSpec((BN, n), lambda i: (i, 0)),
        out_shape=jax.ShapeDtypeStruct((m, n), jnp.float32),
    )(x, w)


def _scale_body(dp_ref, xw_ref, y_ref, dis_ref):
    deg = 1.0 + jnp.sum(dp_ref[...], axis=1)[:, None]
    dis = lax.rsqrt(deg)
    dis_ref[...] = dis
    y_ref[...] = xw_ref[...] * dis


def _tc_scale(degp, xw):
    return pl.pallas_call(
        _scale_body,
        grid=(N // BN,),
        in_specs=[pl.BlockSpec((BN, NW), lambda i: (i, 0)),
                  pl.BlockSpec((BN, H), lambda i: (i, 0))],
        out_specs=[pl.BlockSpec((BN, H), lambda i: (i, 0)),
                   pl.BlockSpec((BN, 1), lambda i: (i, 0))],
        out_shape=[jax.ShapeDtypeStruct((N, H), jnp.float32),
                   jax.ShapeDtypeStruct((N, 1), jnp.float32)],
    )(degp, xw)


def _ln(h, g_ref, be_ref):
    m = jnp.mean(h, axis=-1, keepdims=True)
    c = h - m
    v = jnp.mean(c * c, axis=-1, keepdims=True)
    return c * lax.rsqrt(v + 1e-5) * g_ref[...] + be_ref[...]


def _mid_body(p_ref, y_ref, dis_ref, b_ref, g_ref, be_ref, w_ref,
              x1_ref, y2_ref):
    agg = p_ref[0] + p_ref[1] + y_ref[...]
    h = agg * dis_ref[...] + b_ref[...]
    x1 = jnp.maximum(_ln(h, g_ref, be_ref), 0.0)
    x1_ref[...] = x1
    y2_ref[...] = jnp.dot(x1, w_ref[...],
                          preferred_element_type=jnp.float32) * dis_ref[...]


def _tc_mid(p, y1, dis, b1, g1, be1, w2):
    return pl.pallas_call(
        _mid_body,
        grid=(N // BN,),
        in_specs=[pl.BlockSpec((NC, BN, H), lambda i: (0, i, 0)),
                  pl.BlockSpec((BN, H), lambda i: (i, 0)),
                  pl.BlockSpec((BN, 1), lambda i: (i, 0)),
                  pl.BlockSpec((1, H), lambda i: (0, 0)),
                  pl.BlockSpec((1, H), lambda i: (0, 0)),
                  pl.BlockSpec((1, H), lambda i: (0, 0)),
                  pl.BlockSpec((H, H), lambda i: (0, 0))],
        out_specs=[pl.BlockSpec((BN, H), lambda i: (i, 0)),
                   pl.BlockSpec((BN, H), lambda i: (i, 0))],
        out_shape=[jax.ShapeDtypeStruct((N, H), jnp.float32),
                   jax.ShapeDtypeStruct((N, H), jnp.float32)],
    )(p, y1, dis, b1, g1, be1, w2)


def _final_body(q_ref, y_ref, dis_ref, b_ref, g_ref, be_ref, x1_ref,
                wf_ref, bf_ref, o_ref):
    agg = q_ref[0] + q_ref[1] + y_ref[...]
    h = agg * dis_ref[...] + b_ref[...]
    x2 = jnp.maximum(_ln(h, g_ref, be_ref), 0.0) + x1_ref[...]
    o_ref[...] = jnp.dot(x2, wf_ref[...],
                         preferred_element_type=jnp.float32) + bf_ref[...]


def _tc_final(q, y2, dis, b2, g2, be2, x1, wf, bf):
    return pl.pallas_call(
        _final_body,
        grid=(N // BN,),
        in_specs=[pl.BlockSpec((NC, BN, H), lambda i: (0, i, 0)),
                  pl.BlockSpec((BN, H), lambda i: (i, 0)),
                  pl.BlockSpec((BN, 1), lambda i: (i, 0)),
                  pl.BlockSpec((1, H), lambda i: (0, 0)),
                  pl.BlockSpec((1, H), lambda i: (0, 0)),
                  pl.BlockSpec((1, H), lambda i: (0, 0)),
                  pl.BlockSpec((BN, H), lambda i: (i, 0)),
                  pl.BlockSpec((H, 1), lambda i: (0, 0)),
                  pl.BlockSpec((1, 1), lambda i: (0, 0))],
        out_specs=pl.BlockSpec((BN, 1), lambda i: (i, 0)),
        out_shape=jax.ShapeDtypeStruct((N, 1), jnp.float32),
    )(q, y2, dis, b2, g2, be2, x1, wf, bf)


def kernel(x, edge_index, W1, b1, g1, be1, W2, b2, g2, be2, Wf, bf):
    src_i32 = edge_index[0].astype(jnp.int32)
    dst_i32 = edge_index[1].astype(jnp.int32)
    src3 = src_i32.reshape(NW * NSTG, HS, CS)
    dst3 = dst_i32.reshape(NW * NSTG, HS, CS)
    dstv = jnp.concatenate(
        [dst_i32.reshape(NW, EPW), jnp.full((NW, PAD), N, jnp.int32)],
        axis=1)
    zeros_rows = jnp.zeros((N, H), jnp.float32)
    zeros_cnt = jnp.zeros((CNT,), jnp.float32)

    b1r = b1.reshape(1, H)
    g1r = g1.reshape(1, H)
    be1r = be1.reshape(1, H)
    b2r = b2.reshape(1, H)
    g2r = g2.reshape(1, H)
    be2r = be2.reshape(1, H)
    bfr = bf.reshape(1, 1)

    sc_scatter = _sc_scatter_kernel()
    sc_degree = _sc_degree_kernel()

    degp = sc_degree(dstv, zeros_cnt)[:, :N].T
    xw1 = _tc_matmul(x, W1)
    y1, dis = _tc_scale(degp, xw1)

    p = sc_scatter(y1, src3, dst3, zeros_rows)
    x1, y2 = _tc_mid(p, y1, dis, b1r, g1r, be1r, W2)

    q = sc_scatter(y2, src3, dst3, zeros_rows)
    return _tc_final(q, y2, dis, b2r, g2r, be2r, x1, Wf, bfr)


# speedup vs baseline: 4.0484x; 1.0089x over previous
import functools

import jax
import jax.numpy as jnp
from jax import lax
from jax.experimental import pallas as pl
from jax.experimental.pallas import tpu as pltpu
from jax.experimental.pallas import tpu_sc as plsc

N = 10000
E = 320000
D = 128
H = 128

NC = 2
NS = 16
NW = NC * NS
EPW = E // NW
CS = 80
NCH = 125
NSTG = 5
HS = NCH // NSTG
NBUF = 4
PAD = 240

NV = (EPW + PAD) // 16
CNT = N + 16

ZR = 624
TAIL = N - NS * ZR
TAIL_OFF = NS * ZR


def _striped_copy(src_ref, dst_ref, sid):
    pltpu.sync_copy(src_ref.at[pl.ds(sid * ZR, ZR)],
                    dst_ref.at[pl.ds(sid * ZR, ZR)])

    @pl.when(sid == 0)
    def _():
        pltpu.sync_copy(src_ref.at[pl.ds(TAIL_OFF, TAIL)],
                        dst_ref.at[pl.ds(TAIL_OFF, TAIL)])


@functools.lru_cache(maxsize=None)
def _sc_degree_kernel():
    mesh = plsc.VectorSubcoreMesh(core_axis_name="c", subcore_axis_name="s")
    return functools.partial(
        pl.kernel,
        mesh=mesh,
        out_type=jax.ShapeDtypeStruct((NW, CNT), jnp.float32),
        scratch_types=[
            pltpu.VMEM((NV * 16,), jnp.int32),
            pltpu.VMEM((CNT,), jnp.float32),
        ],
        compiler_params=pltpu.CompilerParams(needs_layout_passes=False),
    )(_sc_degree_body)


def _sc_degree_body(dst_hbm, zeros_hbm, out_hbm, idx_v, cnt_v):
    cid = lax.axis_index("c")
    sid = lax.axis_index("s")
    wid = sid * NC + cid
    pltpu.sync_copy(zeros_hbm, cnt_v)
    pltpu.sync_copy(dst_hbm.at[wid], idx_v)
    ones = jnp.ones((16,), jnp.float32)

    @pl.loop(0, NV)
    def _(r):
        idx = idx_v[pl.ds(r * 16, 16)]
        plsc.addupdate_scatter(cnt_v, [idx], ones)

    pltpu.sync_copy(cnt_v, out_hbm.at[wid])


@functools.lru_cache(maxsize=None)
def _sc_scatter_kernel():
    mesh = plsc.VectorSubcoreMesh(core_axis_name="c", subcore_axis_name="s")
    return functools.partial(
        pl.kernel,
        mesh=mesh,
        out_type=jax.ShapeDtypeStruct((NC, N, H), jnp.float32),
        scratch_types=[
            pltpu.VMEM((HS, CS), jnp.int32),
            pltpu.VMEM((HS, CS), jnp.int32),
        ] + [pltpu.VMEM((CS, H), jnp.float32)] * NBUF
          + [pltpu.SemaphoreType.DMA] * NBUF
          + [pltpu.VMEM_SHARED((N, H), jnp.float32)],
    )(_sc_scatter_body)


def _sc_scatter_body(y_hbm, src_hbm, dst_hbm, zeros_hbm, out_hbm,
                     src_v, dst_v, *rest):
    bufs = rest[:NBUF]
    sems = rest[NBUF:2 * NBUF]
    acc_sh = rest[2 * NBUF]
    cid = lax.axis_index("c")
    sid = lax.axis_index("s")
    wid = sid * NC + cid
    _striped_copy(zeros_hbm, acc_sh, sid)
    plsc.subcore_barrier()

    for s in range(NSTG):
        pid = wid * NSTG + s
        pltpu.sync_copy(src_hbm.at[pid], src_v)
        pltpu.sync_copy(dst_hbm.at[pid], dst_v)
        for b in range(NBUF):
            pltpu.async_copy(y_hbm.at[src_v.at[b]], bufs[b], sems[b])

        @pl.loop(0, HS - 1, step=NBUF)
        def _(j):
            for b in range(NBUF):
                jj = j + b
                buf = bufs[b]
                sem = sems[b]

                pltpu.make_async_copy(y_hbm.at[src_v.at[jj]], buf, sem).wait()
                pltpu.sync_copy(buf, acc_sh.at[dst_v.at[jj]], add=True)

                @pl.when(jj + NBUF < HS)
                def _():
                    pltpu.async_copy(y_hbm.at[src_v.at[jj + NBUF]], buf, sem)

        pltpu.make_async_copy(y_hbm.at[src_v.at[HS - 1]], bufs[0],
                              sems[0]).wait()
        pltpu.sync_copy(bufs[0], acc_sh.at[dst_v.at[HS - 1]], add=True)

    plsc.subcore_barrier()
    _striped_copy(acc_sh, out_hbm.at[cid], sid)


BN = 1000


def _scale_body(dp_ref, x_ref, w_ref, y_ref, dis_ref):
    deg = 1.0 + jnp.sum(dp_ref[...], axis=1)[:, None]
    dis = lax.rsqrt(deg)
    dis_ref[...] = dis
    y_ref[...] = jnp.dot(x_ref[...], w_ref[...],
                         preferred_element_type=jnp.float32) * dis


def _tc_scale(degp, x, w1):
    return pl.pallas_call(
        _scale_body,
        grid=(N // BN,),
        in_specs=[pl.BlockSpec((BN, NW), lambda i: (i, 0)),
                  pl.BlockSpec((BN, D), lambda i: (i, 0)),
                  pl.BlockSpec((D, H), lambda i: (0, 0))],
        out_specs=[pl.BlockSpec((BN, H), lambda i: (i, 0)),
                   pl.BlockSpec((BN, 1), lambda i: (i, 0))],
        out_shape=[jax.ShapeDtypeStruct((N, H), jnp.float32),
                   jax.ShapeDtypeStruct((N, 1), jnp.float32)],
    )(degp, x, w1)


def _ln(h, g_ref, be_ref):
    m = jnp.mean(h, axis=-1, keepdims=True)
    c = h - m
    v = jnp.mean(c * c, axis=-1, keepdims=True)
    return c * lax.rsqrt(v + 1e-5) * g_ref[...] + be_ref[...]


def _mid_body(p_ref, y_ref, dis_ref, b_ref, g_ref, be_ref, w_ref,
              x1_ref, y2_ref):
    agg = p_ref[0] + p_ref[1] + y_ref[...]
    h = agg * dis_ref[...] + b_ref[...]
    x1 = jnp.maximum(_ln(h, g_ref, be_ref), 0.0)
    x1_ref[...] = x1
    y2_ref[...] = jnp.dot(x1, w_ref[...],
                          preferred_element_type=jnp.float32) * dis_ref[...]


def _tc_mid(p, y1, dis, b1, g1, be1, w2):
    return pl.pallas_call(
        _mid_body,
        grid=(N // BN,),
        in_specs=[pl.BlockSpec((NC, BN, H), lambda i: (0, i, 0)),
                  pl.BlockSpec((BN, H), lambda i: (i, 0)),
                  pl.BlockSpec((BN, 1), lambda i: (i, 0)),
                  pl.BlockSpec((1, H), lambda i: (0, 0)),
                  pl.BlockSpec((1, H), lambda i: (0, 0)),
                  pl.BlockSpec((1, H), lambda i: (0, 0)),
                  pl.BlockSpec((H, H), lambda i: (0, 0))],
        out_specs=[pl.BlockSpec((BN, H), lambda i: (i, 0)),
                   pl.BlockSpec((BN, H), lambda i: (i, 0))],
        out_shape=[jax.ShapeDtypeStruct((N, H), jnp.float32),
                   jax.ShapeDtypeStruct((N, H), jnp.float32)],
    )(p, y1, dis, b1, g1, be1, w2)


def _final_body(q_ref, y_ref, dis_ref, b_ref, g_ref, be_ref, x1_ref,
                wf_ref, bf_ref, o_ref):
    agg = q_ref[0] + q_ref[1] + y_ref[...]
    h = agg * dis_ref[...] + b_ref[...]
    x2 = jnp.maximum(_ln(h, g_ref, be_ref), 0.0) + x1_ref[...]
    o_ref[...] = jnp.dot(x2, wf_ref[...],
                         preferred_element_type=jnp.float32) + bf_ref[...]


def _tc_final(q, y2, dis, b2, g2, be2, x1, wf, bf):
    return pl.pallas_call(
        _final_body,
        grid=(N // BN,),
        in_specs=[pl.BlockSpec((NC, BN, H), lambda i: (0, i, 0)),
                  pl.BlockSpec((BN, H), lambda i: (i, 0)),
                  pl.BlockSpec((BN, 1), lambda i: (i, 0)),
                  pl.BlockSpec((1, H), lambda i: (0, 0)),
                  pl.BlockSpec((1, H), lambda i: (0, 0)),
                  pl.BlockSpec((1, H), lambda i: (0, 0)),
                  pl.BlockSpec((BN, H), lambda i: (i, 0)),
                  pl.BlockSpec((H, 1), lambda i: (0, 0)),
                  pl.BlockSpec((1, 1), lambda i: (0, 0))],
        out_specs=pl.BlockSpec((BN, 1), lambda i: (i, 0)),
        out_shape=jax.ShapeDtypeStruct((N, 1), jnp.float32),
    )(q, y2, dis, b2, g2, be2, x1, wf, bf)


def kernel(x, edge_index, W1, b1, g1, be1, W2, b2, g2, be2, Wf, bf):
    src_i32 = edge_index[0].astype(jnp.int32)
    dst_i32 = edge_index[1].astype(jnp.int32)
    src3 = src_i32.reshape(NW * NSTG, HS, CS)
    dst3 = dst_i32.reshape(NW * NSTG, HS, CS)
    dstv = jnp.concatenate(
        [dst_i32.reshape(NW, EPW), jnp.full((NW, PAD), N, jnp.int32)],
        axis=1)
    zeros_rows = jnp.zeros((N, H), jnp.float32)
    zeros_cnt = jnp.zeros((CNT,), jnp.float32)

    b1r = b1.reshape(1, H)
    g1r = g1.reshape(1, H)
    be1r = be1.reshape(1, H)
    b2r = b2.reshape(1, H)
    g2r = g2.reshape(1, H)
    be2r = be2.reshape(1, H)
    bfr = bf.reshape(1, 1)

    sc_scatter = _sc_scatter_kernel()
    sc_degree = _sc_degree_kernel()

    degp = sc_degree(dstv, zeros_cnt)[:, :N].T
    y1, dis = _tc_scale(degp, x, W1)

    p = sc_scatter(y1, src3, dst3, zeros_rows)
    x1, y2 = _tc_mid(p, y1, dis, b1r, g1r, be1r, W2)

    q = sc_scatter(y2, src3, dst3, zeros_rows)
    return _tc_final(q, y2, dis, b2r, g2r, be2r, x1, Wf, bfr)
